# Initial kernel scaffold; baseline (speedup 1.0000x reference)
import jax
import jax.numpy as jnp
from jax.experimental import pallas as pl


def kernel(feat, rois):
    raise NotImplementedError("write your pallas kernel here")


# speedup vs baseline: 52.9203x; 52.9203x over previous
import jax
import jax.numpy as jnp
import numpy as np
from jax import lax
from jax.experimental import pallas as pl
from jax.experimental.pallas import tpu as pltpu
from jax.experimental.pallas import tpu_sc as plsc

_SCALE = 1.0 / 16.0
_P = 7
_S = 2
_D = 5
_H = 64
_W = 64
_C = _D * _P * _P
_R = 512
_NTILES = 32
_CPT = 8
_C0MAX = _C - _CPT
_RB = _R // 16
_HW = _H * _W


def _tile_body(feat_hbm, rois_hbm, ytab_hbm, xtab_hbm, out_hbm,
               feat_v, rois_v, ytab_v, xtab_v, iyd, fyd, ixd, fxd,
               ibv, outbuf, sem0, sem1):
    nc = 2
    wid = lax.axis_index("s") * nc + lax.axis_index("c")
    c0 = jnp.minimum(wid * _CPT, _C0MAX)
    d0 = wid * _CPT - c0

    cwords = _CPT * _HW
    cp0 = pltpu.async_copy(feat_hbm.at[0, pl.ds(c0 * _HW, cwords)],
                           feat_v.at[0], sem0)
    cp1 = pltpu.async_copy(feat_hbm.at[1, pl.ds(c0 * _HW, cwords)],
                           feat_v.at[1], sem1)
    pltpu.sync_copy(rois_hbm, rois_v)
    pltpu.sync_copy(ytab_hbm, ytab_v)
    pltpu.sync_copy(xtab_hbm, xtab_v)

    lane = lax.broadcasted_iota(jnp.int32, (16,), 0)
    ytabv = ytab_v[pl.ds(wid * 16, 16)]
    xtabv = xtab_v[pl.ds(wid * 16, 16)]

    @plsc.parallel_loop(0, _RB)
    def _(rb):
        ridx = rb * 16 + lane
        b_f = plsc.load_gather(rois_v, [ridx * 5])
        rx1 = plsc.load_gather(rois_v, [ridx * 5 + 1])
        ry1 = plsc.load_gather(rois_v, [ridx * 5 + 2])
        rx2 = plsc.load_gather(rois_v, [ridx * 5 + 3])
        ry2 = plsc.load_gather(rois_v, [ridx * 5 + 4])
        ibv[pl.ds(rb * 16, 16)] = b_f.astype(jnp.int32)

        sw = rx1 * _SCALE
        sh = ry1 * _SCALE
        ew = rx2 * _SCALE
        eh = ry2 * _SCALE
        bin_w = jnp.maximum(ew - sw, 0.1) / float(_P)
        bin_h = jnp.maximum(eh - sh, 0.1) / float(_P)

        for p in range(_P):
            for s in range(_S):
                frac = p + (s + 0.5) / _S
                ofi = rb * 224 + p * 32 + s * 16
                off = rb * 448 + p * 64 + s * 32
                yq = sh + frac * bin_h
                vy = jnp.where((yq >= -1.0) & (yq <= float(_H)), 1.0, 0.0)
                yc = jnp.minimum(jnp.maximum(yq, 0.0), float(_H - 1))
                y0 = yc.astype(jnp.int32)
                iyd[pl.ds(ofi, 16)] = y0 * _W
                fyd[pl.ds(off, 16)] = yc - y0.astype(jnp.float32)
                fyd[pl.ds(off + 16, 16)] = vy

                xq = sw + frac * bin_w
                vx = jnp.where((xq >= -1.0) & (xq <= float(_W)), 1.0, 0.0)
                xc = jnp.minimum(jnp.maximum(xq, 0.0), float(_W - 1))
                x0 = xc.astype(jnp.int32)
                ixd[pl.ds(ofi, 16)] = x0
                fxd[pl.ds(off, 16)] = xc - x0.astype(jnp.float32)
                fxd[pl.ds(off + 16, 16)] = vx

    cp0.wait()
    cp1.wait()

    @plsc.parallel_loop(0, _RB)
    def _(rb):
        ridx = rb * 16 + lane
        bvec = ibv[pl.ds(rb * 16, 16)]
        for cl in range(_CPT):
            yb = ytabv[cl]
            xb = xtabv[cl]
            cbase = jnp.minimum(cl + d0, _CPT - 1) * _HW
            acc = jnp.zeros((16,), jnp.float32)
            for s_y in range(_S):
                yo0 = iyd[pl.ds(rb * 224 + yb * 32 + s_y * 16, 16)] + cbase
                yo1 = jnp.minimum(yo0 + _W, cbase + (_H - 1) * _W)
                fo = rb * 448 + yb * 64 + s_y * 32
                ly = fyd[pl.ds(fo, 16)]
                vy = fyd[pl.ds(fo + 16, 16)]
                for s_x in range(_S):
                    xo0 = ixd[pl.ds(rb * 224 + xb * 32 + s_x * 16, 16)]
                    xo1 = jnp.minimum(xo0 + 1, _W - 1)
                    fo2 = rb * 448 + xb * 64 + s_x * 32
                    lx = fxd[pl.ds(fo2, 16)]
                    vx = fxd[pl.ds(fo2 + 16, 16)]
                    v1 = plsc.load_gather(feat_v, [bvec, yo0 + xo0])
                    v2 = plsc.load_gather(feat_v, [bvec, yo0 + xo1])
                    v3 = plsc.load_gather(feat_v, [bvec, yo1 + xo0])
                    v4 = plsc.load_gather(feat_v, [bvec, yo1 + xo1])
                    top = v1 + lx * (v2 - v1)
                    bot = v3 + lx * (v4 - v3)
                    val = top + ly * (bot - top)
                    acc = acc + val * (vy * vx)
            clv = jnp.full((16,), cl, jnp.int32)
            plsc.store_scatter(outbuf, [ridx, clv], acc * (1.0 / (_S * _S)))

    col0 = pl.multiple_of(wid * _CPT, _CPT)
    pltpu.sync_copy(outbuf, out_hbm.at[:, pl.ds(col0, _CPT)])


@jax.jit
def _psroi_sc(feat_flat, rois_flat, ytab, xtab):
    mesh = plsc.VectorSubcoreMesh(core_axis_name="c", subcore_axis_name="s")
    f = pl.kernel(
        _tile_body,
        mesh=mesh,
        out_type=jax.ShapeDtypeStruct((_R, _NTILES * _CPT), jnp.float32),
        compiler_params=pltpu.CompilerParams(needs_layout_passes=False,
                                             use_tc_tiling_on_sc=False),
        scratch_types=[
            pltpu.VMEM((2, _CPT * _HW), jnp.float32),
            pltpu.VMEM((_R * 5,), jnp.float32),
            pltpu.VMEM((_NTILES * 16,), jnp.int32),
            pltpu.VMEM((_NTILES * 16,), jnp.int32),
            pltpu.VMEM((_RB * 224,), jnp.int32),
            pltpu.VMEM((_RB * 448,), jnp.float32),
            pltpu.VMEM((_RB * 224,), jnp.int32),
            pltpu.VMEM((_RB * 448,), jnp.float32),
            pltpu.VMEM((_R,), jnp.int32),
            pltpu.VMEM((_R, _CPT), jnp.float32),
            pltpu.SemaphoreType.DMA,
            pltpu.SemaphoreType.DMA,
        ],
    )
    return f(feat_flat, rois_flat, ytab, xtab)


def _mk_tabs():
    yt = np.zeros(_NTILES * 16, np.int32)
    xt = np.zeros(_NTILES * 16, np.int32)
    for t in range(_NTILES):
        for j in range(16):
            ch = min(t * _CPT + j, _C - 1)
            yt[t * 16 + j] = (ch % (_P * _P)) // _P
            xt[t * 16 + j] = ch % _P
    return yt, xt


_YTAB, _XTAB = _mk_tabs()


def kernel(feat, rois):
    n, c, h, w = feat.shape
    feat_flat = feat.reshape(n, c * h * w)
    out = _psroi_sc(feat_flat, rois.reshape(-1), _YTAB, _XTAB)
    return out[:, :_C].reshape(_R, _D, _P, _P)


# speedup vs baseline: 54.7818x; 1.0352x over previous
import jax
import jax.numpy as jnp
import numpy as np
from jax import lax
from jax.experimental import pallas as pl
from jax.experimental.pallas import tpu as pltpu
from jax.experimental.pallas import tpu_sc as plsc

_SCALE = 1.0 / 16.0
_P = 7
_S = 2
_D = 5
_H = 64
_W = 64
_C = _D * _P * _P
_R = 512
_NTILES = 32
_CPT = 8
_C0MAX = _C - _CPT
_RB = _R // 16
_HW = _H * _W


def _tile_body(feat_hbm, rois_hbm, ytab_hbm, xtab_hbm, out_hbm,
               feat_v, rois_v, ytab_v, xtab_v, iyd, fyd, ixd, fxd,
               ibv, outbuf, sem0, sem1):
    nc = 2
    wid = lax.axis_index("s") * nc + lax.axis_index("c")
    c0 = jnp.minimum(wid * _CPT, _C0MAX)
    d0 = wid * _CPT - c0

    cp0 = pltpu.async_copy(feat_hbm.at[0, pl.ds(c0, _CPT)], feat_v.at[0], sem0)
    cp1 = pltpu.async_copy(feat_hbm.at[1, pl.ds(c0, _CPT)], feat_v.at[1], sem1)
    pltpu.sync_copy(rois_hbm, rois_v)
    pltpu.sync_copy(ytab_hbm, ytab_v)
    pltpu.sync_copy(xtab_hbm, xtab_v)

    lane = lax.broadcasted_iota(jnp.int32, (16,), 0)
    ytabv = ytab_v[pl.ds(wid * 16, 16)]
    xtabv = xtab_v[pl.ds(wid * 16, 16)]

    @plsc.parallel_loop(0, _RB)
    def _(rb):
        ridx = rb * 16 + lane
        b_f = plsc.load_gather(rois_v, [ridx * 5])
        rx1 = plsc.load_gather(rois_v, [ridx * 5 + 1])
        ry1 = plsc.load_gather(rois_v, [ridx * 5 + 2])
        rx2 = plsc.load_gather(rois_v, [ridx * 5 + 3])
        ry2 = plsc.load_gather(rois_v, [ridx * 5 + 4])
        ibv[pl.ds(rb * 16, 16)] = b_f.astype(jnp.int32)

        sw = rx1 * _SCALE
        sh = ry1 * _SCALE
        ew = rx2 * _SCALE
        eh = ry2 * _SCALE
        bin_w = jnp.maximum(ew - sw, 0.1) / float(_P)
        bin_h = jnp.maximum(eh - sh, 0.1) / float(_P)

        for p in range(_P):
            for s in range(_S):
                frac = p + (s + 0.5) / _S
                ofi = rb * 224 + p * 32 + s * 16
                off = rb * 448 + p * 64 + s * 32
                yq = sh + frac * bin_h
                vy = jnp.where((yq >= -1.0) & (yq <= float(_H)), 1.0, 0.0)
                yc = jnp.minimum(jnp.maximum(yq, 0.0), float(_H - 1))
                y0 = yc.astype(jnp.int32)
                iyd[pl.ds(ofi, 16)] = y0
                fyd[pl.ds(off, 16)] = yc - y0.astype(jnp.float32)
                fyd[pl.ds(off + 16, 16)] = vy

                xq = sw + frac * bin_w
                vx = jnp.where((xq >= -1.0) & (xq <= float(_W)), 1.0, 0.0)
                xc = jnp.minimum(jnp.maximum(xq, 0.0), float(_W - 1))
                x0 = xc.astype(jnp.int32)
                ixd[pl.ds(ofi, 16)] = x0
                fxd[pl.ds(off, 16)] = xc - x0.astype(jnp.float32)
                fxd[pl.ds(off + 16, 16)] = vx

    cp0.wait()
    cp1.wait()

    @plsc.parallel_loop(0, _RB)
    def _(rb):
        ridx = rb * 16 + lane
        bvec = ibv[pl.ds(rb * 16, 16)]
        for cl in range(_CPT):
            yb = ytabv[cl]
            xb = xtabv[cl]
            cv = jnp.zeros((16,), jnp.int32) + jnp.minimum(cl + d0, _CPT - 1)
            acc = jnp.zeros((16,), jnp.float32)
            for s_y in range(_S):
                yo0 = iyd[pl.ds(rb * 224 + yb * 32 + s_y * 16, 16)]
                yo1 = jnp.minimum(yo0 + 1, _H - 1)
                fo = rb * 448 + yb * 64 + s_y * 32
                ly = fyd[pl.ds(fo, 16)]
                vy = fyd[pl.ds(fo + 16, 16)]
                for s_x in range(_S):
                    xo0 = ixd[pl.ds(rb * 224 + xb * 32 + s_x * 16, 16)]
                    xo1 = jnp.minimum(xo0 + 1, _W - 1)
                    fo2 = rb * 448 + xb * 64 + s_x * 32
                    lx = fxd[pl.ds(fo2, 16)]
                    vx = fxd[pl.ds(fo2 + 16, 16)]
                    v1 = plsc.load_gather(feat_v, [bvec, cv, yo0, xo0])
                    v2 = plsc.load_gather(feat_v, [bvec, cv, yo0, xo1])
                    v3 = plsc.load_gather(feat_v, [bvec, cv, yo1, xo0])
                    v4 = plsc.load_gather(feat_v, [bvec, cv, yo1, xo1])
                    top = v1 + lx * (v2 - v1)
                    bot = v3 + lx * (v4 - v3)
                    val = top + ly * (bot - top)
                    acc = acc + val * (vy * vx)
            clv = jnp.full((16,), cl, jnp.int32)
            plsc.store_scatter(outbuf, [ridx, clv], acc * (1.0 / (_S * _S)))

    col0 = pl.multiple_of(wid * _CPT, _CPT)
    pltpu.sync_copy(outbuf, out_hbm.at[:, pl.ds(col0, _CPT)])


@jax.jit
def _psroi_sc(feat_flat, rois_flat, ytab, xtab):
    mesh = plsc.VectorSubcoreMesh(core_axis_name="c", subcore_axis_name="s")
    f = pl.kernel(
        _tile_body,
        mesh=mesh,
        out_type=jax.ShapeDtypeStruct((_R, _NTILES * _CPT), jnp.float32),
        compiler_params=pltpu.CompilerParams(needs_layout_passes=False,
                                             use_tc_tiling_on_sc=False),
        scratch_types=[
            pltpu.VMEM((2, _CPT, _H, _W), jnp.float32),
            pltpu.VMEM((_R * 5,), jnp.float32),
            pltpu.VMEM((_NTILES * 16,), jnp.int32),
            pltpu.VMEM((_NTILES * 16,), jnp.int32),
            pltpu.VMEM((_RB * 224,), jnp.int32),
            pltpu.VMEM((_RB * 448,), jnp.float32),
            pltpu.VMEM((_RB * 224,), jnp.int32),
            pltpu.VMEM((_RB * 448,), jnp.float32),
            pltpu.VMEM((_R,), jnp.int32),
            pltpu.VMEM((_R, _CPT), jnp.float32),
            pltpu.SemaphoreType.DMA,
            pltpu.SemaphoreType.DMA,
        ],
    )
    return f(feat_flat, rois_flat, ytab, xtab)


def _mk_tabs():
    yt = np.zeros(_NTILES * 16, np.int32)
    xt = np.zeros(_NTILES * 16, np.int32)
    for t in range(_NTILES):
        for j in range(16):
            ch = min(t * _CPT + j, _C - 1)
            yt[t * 16 + j] = (ch % (_P * _P)) // _P
            xt[t * 16 + j] = ch % _P
    return yt, xt


_YTAB, _XTAB = _mk_tabs()


def kernel(feat, rois):
    out = _psroi_sc(feat, rois.reshape(-1), _YTAB, _XTAB)
    return out[:, :_C].reshape(_R, _D, _P, _P)
